# baseline (speedup 1.0000x reference)
import functools

import jax
import jax.numpy as jnp
from jax import lax
from jax.experimental import pallas as pl
from jax.experimental.pallas import tpu as pltpu
from jax.experimental.pallas import tpu_sc as plsc

N = 10000
E = 320000
F = 128
NPAD = 10240
RB = 1024
NW = 32
EPW = E // NW
CH = 80
NCH = EPW // CH
KD = 5
CHA = 80
NCHA = EPW // CHA
SROWS = NPAD // 16

_mesh = plsc.VectorSubcoreMesh(core_axis_name="c", subcore_axis_name="s",
                               num_cores=2, num_subcores=16)
_sc_params = pltpu.CompilerParams(needs_layout_passes=False)


@functools.partial(
    pl.kernel,
    out_type=jax.ShapeDtypeStruct((2, NPAD, 16), jnp.float32),
    mesh=_mesh,
    compiler_params=_sc_params,
    scratch_types=[
        pltpu.VMEM_SHARED((NPAD, 16), jnp.float32),
        pltpu.VMEM((NCH, CH), jnp.int32),
        pltpu.VMEM((NCH, CH), jnp.float32),
        pltpu.VMEM((KD, CH, 16), jnp.float32),
        pltpu.SemaphoreType.DMA,
    ],
)
def _deg_kernel(dst_hbm, ew_hbm, out_hbm, acc_sh, dst2d, ew2d, bufs, sem):
    c = lax.axis_index("c")
    s = lax.axis_index("s")
    wid = s * 2 + c
    zero16 = jnp.zeros((16,), jnp.float32)
    iota16 = lax.iota(jnp.int32, 16)

    for b in range(KD):
        for i in range(CH):
            bufs[b, i, :] = zero16

    @pl.loop(0, SROWS // CH)
    def _zero_acc(k):
        pltpu.sync_copy(bufs.at[0], acc_sh.at[pl.ds(s * SROWS + k * CH, CH), :])

    pltpu.sync_copy(dst_hbm.at[wid], dst2d)
    pltpu.sync_copy(ew_hbm.at[wid], ew2d)
    plsc.subcore_barrier()

    @pl.loop(0, NCH // KD)
    def _round(ro):
        cps = []
        for b in range(KD):
            ci = ro * KD + b
            for g in range(CH // 16):
                ew16 = plsc.load_gather(ew2d, [iota16 * 0 + ci, iota16 + g * 16])
                plsc.store_scatter(bufs.at[b], [iota16 + g * 16, iota16 * 0], ew16)
            cps.append(pltpu.async_copy(
                bufs.at[b], acc_sh.at[dst2d.at[ci]], sem, add=True))
        for cp in cps:
            cp.wait()

    plsc.subcore_barrier()

    @pl.loop(0, SROWS // CH)
    def _drain(k):
        r0 = s * SROWS + k * CH
        pltpu.sync_copy(acc_sh.at[pl.ds(r0, CH), :], out_hbm.at[c, pl.ds(r0, CH), :])


@functools.partial(
    pl.kernel,
    out_type=jax.ShapeDtypeStruct((2, NPAD, F), jnp.float32),
    mesh=_mesh,
    compiler_params=_sc_params,
    scratch_types=[
        pltpu.VMEM_SHARED((NPAD, F), jnp.float32),
        pltpu.VMEM((CHA, F), jnp.float32),
        pltpu.VMEM((CHA, F), jnp.float32),
        pltpu.VMEM((CHA, F), jnp.float32),
        pltpu.VMEM((CHA, F), jnp.float32),
        pltpu.VMEM((2, CHA), jnp.int32),
        pltpu.VMEM((CHA,), jnp.int32),
        pltpu.VMEM((CHA,), jnp.int32),
        pltpu.VMEM((2, CHA), jnp.float32),
        pltpu.SemaphoreType.DMA,
        pltpu.SemaphoreType.DMA,
        pltpu.SemaphoreType.DMA,
        pltpu.SemaphoreType.DMA,
    ],
)
def _agg_kernel(y_hbm, src_hbm, dst_hbm, ew_hbm, out_hbm,
                acc_sh, rows0, rows1, outa, outbb, srcb, dstb0, dstb1, ewb,
                gsem0, gsem1, ssem0, ssem1):
    c = lax.axis_index("c")
    s = lax.axis_index("s")
    wid = s * 2 + c
    zero16 = jnp.zeros((16,), jnp.float32)
    iota16 = lax.iota(jnp.int32, 16)
    bufs = (rows0, rows1)
    outs = (outa, outbb)
    dsts = (dstb0, dstb1)
    gsems = (gsem0, gsem1)
    ssems = (ssem0, ssem1)

    for i in range(CHA):
        for v in range(F // 16):
            rows0[i, pl.ds(v * 16, 16)] = zero16

    @pl.loop(0, SROWS // CHA)
    def _zero_acc(k):
        pltpu.sync_copy(rows0, acc_sh.at[pl.ds(s * SROWS + k * CHA, CHA), :])

    plsc.subcore_barrier()

    def _stage(ci, b):
        pltpu.sync_copy(src_hbm.at[wid, ci], srcb.at[b])
        pltpu.sync_copy(ew_hbm.at[wid, ci], ewb.at[b])
        pltpu.async_copy(y_hbm.at[srcb.at[b]], bufs[b], gsems[b])

    def _scat_wait(b):
        pltpu.make_async_copy(outs[b], acc_sh.at[dsts[b]],
                              ssems[b]).wait()

    def _consume(cur, b):
        pltpu.sync_copy(dst_hbm.at[pl.ds(wid * EPW + cur * CHA, CHA)],
                        dsts[b])
        pltpu.make_async_copy(y_hbm.at[srcb.at[b]], bufs[b], gsems[b]).wait()

        @plsc.parallel_loop(0, CHA - 1, unroll=4)
        def _scale(e):
            rowid = iota16 * 0 + e
            wb = plsc.load_gather(ewb, [iota16 * 0 + b, rowid])
            for v in range(F // 16):
                colv = iota16 + v * 16
                vals = plsc.load_gather(bufs[b], [rowid, colv])
                plsc.store_scatter(outs[b], [rowid, colv], vals * wb)

        eL = CHA - 1
        wbL = plsc.load_gather(ewb, [iota16 * 0 + b, iota16 * 0 + eL])
        for v in range(F // 16):
            sl = pl.ds(v * 16, 16)
            outs[b][eL, sl] = bufs[b][eL, sl] * wbL

    _stage(0, 0)
    _stage(1, 1)

    @pl.loop(0, (NCHA - 1) // 2)
    def _round(ro):
        cur0 = ro * 2
        _consume(cur0, 0)
        d0 = pltpu.async_copy(outs[0], acc_sh.at[dsts[0]], ssems[0],
                              add=True)
        _stage(cur0 + 2, 0)
        _consume(cur0 + 1, 1)
        d0.wait()
        d1 = pltpu.async_copy(outs[1], acc_sh.at[dsts[1]], ssems[1],
                              add=True)
        d1.wait()
        nxt = cur0 + 3

        @pl.when(nxt < NCHA)
        def _refill():
            _stage(nxt, 1)

    _consume(NCHA - 1, 0)
    dL = pltpu.async_copy(outs[0], acc_sh.at[dsts[0]], ssems[0], add=True)
    dL.wait()
    plsc.subcore_barrier()

    @pl.loop(0, SROWS // CH)
    def _drain(k):
        r0 = s * SROWS + k * CH
        pltpu.sync_copy(acc_sh.at[pl.ds(r0, CH), :], out_hbm.at[c, pl.ds(r0, CH), :])


def _prep_body(x_ref, deg2_ref, q_ref, wz_ref, uz_ref, bz_ref, wr_ref,
               ur_ref, br_ref, wh_ref, uh_ref, bh_ref, y_ref):
    Q = q_ref[...]
    z = jax.nn.sigmoid(wz_ref[...] @ Q + uz_ref[...] @ Q + bz_ref[...])
    r = jax.nn.sigmoid(wr_ref[...] @ Q + ur_ref[...] @ Q + br_ref[...])
    hc = jnp.tanh(wh_ref[...] @ Q + uh_ref[...] @ (r * Q) + bh_ref[...])
    W = (1.0 - z) * Q + z * hc
    deg = 1.0 + jnp.sum(deg2_ref[...], axis=1, keepdims=True)
    dinv = lax.rsqrt(deg)
    y_ref[...] = jnp.dot(x_ref[...], W, preferred_element_type=jnp.float32) * dinv


def _prep_call(xpad, deg2, Q, Wz, Uz, bz, Wr, Ur, br, Wh, Uh, bh):
    g = NPAD // RB
    pspec = pl.BlockSpec((F, F), lambda i: (0, 0))
    return pl.pallas_call(
        _prep_body,
        grid=(g,),
        in_specs=[pl.BlockSpec((RB, F), lambda i: (i, 0)),
                  pl.BlockSpec((RB, 32), lambda i: (i, 0))] + [pspec] * 10,
        out_specs=pl.BlockSpec((RB, F), lambda i: (i, 0)),
        out_shape=jax.ShapeDtypeStruct((NPAD, F), jnp.float32),
    )(xpad, deg2, Q, Wz, Uz, bz, Wr, Ur, br, Wh, Uh, bh)


def _final_body(t0_ref, t1_ref, y_ref, deg2_ref, lw_ref, lb_ref, o_ref):
    deg = 1.0 + jnp.sum(deg2_ref[...], axis=1, keepdims=True)
    dinv = lax.rsqrt(deg)
    agg = (t0_ref[...] + t1_ref[...] + y_ref[...]) * dinv
    h = jnp.tanh(agg)
    o_ref[...] = jnp.sum(h * lw_ref[...], axis=1, keepdims=True) + lb_ref[0, 0]


def _final_call(t0, t1, y, deg2, lw, lb):
    g = NPAD // RB
    return pl.pallas_call(
        _final_body,
        grid=(g,),
        in_specs=[pl.BlockSpec((RB, F), lambda i: (i, 0))] * 3 +
                 [pl.BlockSpec((RB, 32), lambda i: (i, 0)),
                  pl.BlockSpec((1, F), lambda i: (0, 0)),
                  pl.BlockSpec((1, 1), lambda i: (0, 0))],
        out_specs=pl.BlockSpec((RB, 1), lambda i: (i, 0)),
        out_shape=jax.ShapeDtypeStruct((NPAD, 1), jnp.float32),
    )(t0, t1, y, deg2, lw, lb)


def kernel(x, edge_index, edge_weight, initial_weight, Wz, Uz, bz, Wr, Ur,
           br, Wh, Uh, bh, lin_w, lin_b):
    src = edge_index[0]
    dst = edge_index[1]
    ew = edge_weight

    xpad = jnp.zeros((NPAD, F), jnp.float32).at[:N].set(x)
    deg_parts = _deg_kernel(dst.reshape(NW, NCH, CH), ew.reshape(NW, NCH, CH))
    deg2 = jnp.swapaxes(deg_parts, 0, 1).reshape(NPAD, 32)
    y = _prep_call(xpad, deg2, initial_weight, Wz, Uz, bz, Wr, Ur, br,
                   Wh, Uh, bh)
    t_parts = _agg_kernel(y, src.reshape(NW, NCHA, CHA), dst,
                          ew.reshape(NW, NCHA, CHA))
    out = _final_call(t_parts[0], t_parts[1], y, deg2, lin_w,
                      lin_b.reshape(1, 1))
    return out[:N]


# speedup vs baseline: 1.2635x; 1.2635x over previous
import functools

import jax
import jax.numpy as jnp
from jax import lax
from jax.experimental import pallas as pl
from jax.experimental.pallas import tpu as pltpu
from jax.experimental.pallas import tpu_sc as plsc

N = 10000
E = 320000
F = 128
NPAD = 10240
RB = 1024
NW = 32
EPW = E // NW
CH = 80
NCH = EPW // CH
KD = 5
CHA = 80
NCHA = EPW // CHA
SROWS = NPAD // 16

_mesh = plsc.VectorSubcoreMesh(core_axis_name="c", subcore_axis_name="s",
                               num_cores=2, num_subcores=16)
_sc_params = pltpu.CompilerParams(needs_layout_passes=False)


@functools.partial(
    pl.kernel,
    out_type=jax.ShapeDtypeStruct((2, NPAD, 16), jnp.float32),
    mesh=_mesh,
    compiler_params=_sc_params,
    scratch_types=[
        pltpu.VMEM_SHARED((NPAD, 16), jnp.float32),
        pltpu.VMEM((NCH, CH), jnp.int32),
        pltpu.VMEM((NCH, CH), jnp.float32),
        pltpu.VMEM((KD, CH, 16), jnp.float32),
        pltpu.SemaphoreType.DMA,
    ],
)
def _deg_kernel(dst_hbm, ew_hbm, out_hbm, acc_sh, dst2d, ew2d, bufs, sem):
    c = lax.axis_index("c")
    s = lax.axis_index("s")
    wid = s * 2 + c
    zero16 = jnp.zeros((16,), jnp.float32)
    iota16 = lax.iota(jnp.int32, 16)

    for b in range(KD):
        for i in range(CH):
            bufs[b, i, :] = zero16

    @pl.loop(0, SROWS // CH)
    def _zero_acc(k):
        pltpu.sync_copy(bufs.at[0], acc_sh.at[pl.ds(s * SROWS + k * CH, CH), :])

    pltpu.sync_copy(dst_hbm.at[wid], dst2d)
    pltpu.sync_copy(ew_hbm.at[wid], ew2d)
    plsc.subcore_barrier()

    @pl.loop(0, NCH // KD)
    def _round(ro):
        cps = []
        for b in range(KD):
            ci = ro * KD + b
            for g in range(CH // 16):
                ew16 = plsc.load_gather(ew2d, [iota16 * 0 + ci, iota16 + g * 16])
                plsc.store_scatter(bufs.at[b], [iota16 + g * 16, iota16 * 0], ew16)
            cps.append(pltpu.async_copy(
                bufs.at[b], acc_sh.at[dst2d.at[ci]], sem, add=True))
        for cp in cps:
            cp.wait()

    plsc.subcore_barrier()

    @pl.loop(0, SROWS // CH)
    def _drain(k):
        r0 = s * SROWS + k * CH
        pltpu.sync_copy(acc_sh.at[pl.ds(r0, CH), :], out_hbm.at[c, pl.ds(r0, CH), :])


@functools.partial(
    pl.kernel,
    out_type=jax.ShapeDtypeStruct((2, NPAD, F), jnp.float32),
    mesh=_mesh,
    compiler_params=_sc_params,
    scratch_types=[
        pltpu.VMEM_SHARED((NPAD, F), jnp.float32),
        pltpu.VMEM((CHA, F), jnp.float32),
        pltpu.VMEM((CHA, F), jnp.float32),
        pltpu.VMEM((CHA, F), jnp.float32),
        pltpu.VMEM((NCHA, CHA), jnp.int32),
        pltpu.VMEM((2, 2, CHA), jnp.int32),
        pltpu.SemaphoreType.DMA,
        pltpu.SemaphoreType.DMA,
    ],
)
def _agg_kernel(y_hbm, rec_hbm, dst_hbm, out_hbm,
                acc_sh, rows0, rows1, outb, dst2d, recb, sem0, sem1):
    c = lax.axis_index("c")
    s = lax.axis_index("s")
    wid = s * 2 + c
    zero16 = jnp.zeros((16,), jnp.float32)
    iota16 = lax.iota(jnp.int32, 16)
    bufs = (rows0, rows1)
    sems = (sem0, sem1)

    for i in range(CHA):
        for v in range(F // 16):
            rows0[i, pl.ds(v * 16, 16)] = zero16

    @pl.loop(0, SROWS // CHA)
    def _zero_acc(k):
        pltpu.sync_copy(rows0, acc_sh.at[pl.ds(s * SROWS + k * CHA, CHA), :])

    pltpu.sync_copy(dst_hbm.at[wid], dst2d)
    plsc.subcore_barrier()

    def _stage(ci, b):
        pltpu.sync_copy(rec_hbm.at[wid * NCHA + ci], recb.at[b])
        pltpu.async_copy(y_hbm.at[recb.at[b, 0]], bufs[b], sems[b])

    def _consume(cur, b):
        pltpu.make_async_copy(y_hbm.at[recb.at[b, 0]], bufs[b], sems[b]).wait()

        @plsc.parallel_loop(0, CHA - 1, unroll=4)
        def _scale(e):
            rowid = iota16 * 0 + e
            wbi = plsc.load_gather(recb, [iota16 * 0 + b, iota16 * 0 + 1, rowid])
            wb = plsc.bitcast(wbi, jnp.float32)
            for v in range(F // 16):
                colv = iota16 + v * 16
                vals = plsc.load_gather(bufs[b], [rowid, colv])
                plsc.store_scatter(outb, [rowid, colv], vals * wb)

        eL = CHA - 1
        wbiL = plsc.load_gather(recb, [iota16 * 0 + b, iota16 * 0 + 1,
                                       iota16 * 0 + eL])
        wbL = plsc.bitcast(wbiL, jnp.float32)
        for v in range(F // 16):
            sl = pl.ds(v * 16, 16)
            outb[eL, sl] = bufs[b][eL, sl] * wbL

        pltpu.sync_copy(outb, acc_sh.at[dst2d.at[cur]], add=True)

    _stage(0, 0)
    _stage(1, 1)

    @pl.loop(0, (NCHA - 1) // 2)
    def _round(ro):
        for b in range(2):
            cur = ro * 2 + b
            _consume(cur, b)
            nxt = cur + 2

            @pl.when(nxt < NCHA)
            def _refill():
                _stage(nxt, b)

    _consume(NCHA - 1, 0)
    plsc.subcore_barrier()

    @pl.loop(0, SROWS // CH)
    def _drain(k):
        r0 = s * SROWS + k * CH
        pltpu.sync_copy(acc_sh.at[pl.ds(r0, CH), :], out_hbm.at[c, pl.ds(r0, CH), :])


def _prep_body(x_ref, deg2_ref, q_ref, wz_ref, uz_ref, bz_ref, wr_ref,
               ur_ref, br_ref, wh_ref, uh_ref, bh_ref, y_ref):
    Q = q_ref[...]
    z = jax.nn.sigmoid(wz_ref[...] @ Q + uz_ref[...] @ Q + bz_ref[...])
    r = jax.nn.sigmoid(wr_ref[...] @ Q + ur_ref[...] @ Q + br_ref[...])
    hc = jnp.tanh(wh_ref[...] @ Q + uh_ref[...] @ (r * Q) + bh_ref[...])
    W = (1.0 - z) * Q + z * hc
    deg = 1.0 + jnp.sum(deg2_ref[...], axis=1, keepdims=True)
    dinv = lax.rsqrt(deg)
    y_ref[...] = jnp.dot(x_ref[...], W, preferred_element_type=jnp.float32) * dinv


def _prep_call(xpad, deg2, Q, Wz, Uz, bz, Wr, Ur, br, Wh, Uh, bh):
    g = NPAD // RB
    pspec = pl.BlockSpec((F, F), lambda i: (0, 0))
    return pl.pallas_call(
        _prep_body,
        grid=(g,),
        in_specs=[pl.BlockSpec((RB, F), lambda i: (i, 0)),
                  pl.BlockSpec((RB, 32), lambda i: (i, 0))] + [pspec] * 10,
        out_specs=pl.BlockSpec((RB, F), lambda i: (i, 0)),
        out_shape=jax.ShapeDtypeStruct((NPAD, F), jnp.float32),
    )(xpad, deg2, Q, Wz, Uz, bz, Wr, Ur, br, Wh, Uh, bh)


def _final_body(t0_ref, t1_ref, y_ref, deg2_ref, lw_ref, lb_ref, o_ref):
    deg = 1.0 + jnp.sum(deg2_ref[...], axis=1, keepdims=True)
    dinv = lax.rsqrt(deg)
    agg = (t0_ref[...] + t1_ref[...] + y_ref[...]) * dinv
    h = jnp.tanh(agg)
    o_ref[...] = jnp.sum(h * lw_ref[...], axis=1, keepdims=True) + lb_ref[0, 0]


def _final_call(t0, t1, y, deg2, lw, lb):
    g = NPAD // RB
    return pl.pallas_call(
        _final_body,
        grid=(g,),
        in_specs=[pl.BlockSpec((RB, F), lambda i: (i, 0))] * 3 +
                 [pl.BlockSpec((RB, 32), lambda i: (i, 0)),
                  pl.BlockSpec((1, F), lambda i: (0, 0)),
                  pl.BlockSpec((1, 1), lambda i: (0, 0))],
        out_specs=pl.BlockSpec((RB, 1), lambda i: (i, 0)),
        out_shape=jax.ShapeDtypeStruct((NPAD, 1), jnp.float32),
    )(t0, t1, y, deg2, lw, lb)


def kernel(x, edge_index, edge_weight, initial_weight, Wz, Uz, bz, Wr, Ur,
           br, Wh, Uh, bh, lin_w, lin_b):
    src = edge_index[0]
    dst = edge_index[1]
    ew = edge_weight

    xpad = jnp.zeros((NPAD, F), jnp.float32).at[:N].set(x)
    deg_parts = _deg_kernel(dst.reshape(NW, NCH, CH), ew.reshape(NW, NCH, CH))
    deg2 = jnp.swapaxes(deg_parts, 0, 1).reshape(NPAD, 32)
    y = _prep_call(xpad, deg2, initial_weight, Wz, Uz, bz, Wr, Ur, br,
                   Wh, Uh, bh)
    rec = jnp.stack(
        [src.reshape(NW * NCHA, CHA),
         jax.lax.bitcast_convert_type(ew, jnp.int32).reshape(NW * NCHA, CHA)],
        axis=1)
    t_parts = _agg_kernel(y, rec, dst.reshape(NW, NCHA, CHA))
    out = _final_call(t_parts[0], t_parts[1], y, deg2, lin_w,
                      lin_b.reshape(1, 1))
    return out[:N]
